# Initial kernel scaffold; baseline (speedup 1.0000x reference)
import jax
import jax.numpy as jnp
from jax.experimental import pallas as pl


def kernel(x, edge_index, W, b):
    raise NotImplementedError("write your pallas kernel here")


# speedup vs baseline: 7.7370x; 7.7370x over previous
import functools

import jax
import jax.numpy as jnp
from jax import lax
from jax.experimental import pallas as pl
from jax.experimental.pallas import tpu as pltpu, tpu_sc as plsc

N = 10000
E = 320000
D = 128
NP = 10112
NR = NP // 128
NC = 2
NS = 16
NW = NC * NS
EP = 327680
EPW = EP // NW
CHUNK = 128
CH = EPW // CHUNK
RPT = NP // NS

_mesh = plsc.VectorSubcoreMesh(core_axis_name="c", subcore_axis_name="s")


@functools.partial(
    pl.kernel,
    mesh=_mesh,
    out_type=jax.ShapeDtypeStruct((NC * 2 * NP,), jnp.float32),
    scratch_types=[
        pltpu.VMEM((CH, CHUNK), jnp.int32),
        pltpu.VMEM((CH, CHUNK), jnp.int32),
        pltpu.VMEM((CHUNK,), jnp.float32),
        pltpu.VMEM((NP,), jnp.float32),
        pltpu.VMEM_SHARED((2 * NP,), jnp.float32),
    ],
)
def _deg_kernel(src_h, dst_h, out_h, sidx, didx, ones_v, stage_v, deg_sh):
    cid = lax.axis_index("c")
    sid = lax.axis_index("s")
    wid = sid * NC + cid

    def _fill_ones(i, _):
        ones_v[pl.ds(i * 16, 16)] = jnp.ones((16,), jnp.float32)
        return 0

    lax.fori_loop(0, CHUNK // 16, _fill_ones, 0)

    def _fill_zero(i, _):
        stage_v[pl.ds(i * 16, 16)] = jnp.zeros((16,), jnp.float32)
        return 0

    lax.fori_loop(0, NP // 16, _fill_zero, 0)

    pltpu.sync_copy(src_h.at[wid], sidx)
    pltpu.sync_copy(dst_h.at[wid], didx)

    @pl.when(sid == 0)
    def _():
        pltpu.sync_copy(stage_v, deg_sh.at[pl.ds(0, NP)])

    @pl.when(sid == 1)
    def _():
        pltpu.sync_copy(stage_v, deg_sh.at[pl.ds(NP, NP)])

    plsc.subcore_barrier()

    def _accum(j, _):
        pltpu.sync_copy(ones_v, deg_sh.at[sidx.at[j]], add=True)
        pltpu.sync_copy(ones_v, deg_sh.at[didx.at[j]], add=True)
        return 0

    lax.fori_loop(0, CH, _accum, 0)

    plsc.subcore_barrier()

    @pl.when(sid == 0)
    def _():
        pltpu.sync_copy(deg_sh.at[pl.ds(0, NP)], stage_v)
        pltpu.sync_copy(stage_v, out_h.at[pl.ds(cid * 2 * NP, NP)])

    @pl.when(sid == 1)
    def _():
        pltpu.sync_copy(deg_sh.at[pl.ds(NP, NP)], stage_v)
        pltpu.sync_copy(stage_v, out_h.at[pl.ds(cid * 2 * NP + NP, NP)])


def _norm_body(dp_ref, ns_ref, nd_ref):
    deg_out = dp_ref[0, :] + dp_ref[2, :]
    deg_in = dp_ref[1, :] + dp_ref[3, :]
    ns_ref[...] = jnp.where(
        deg_out > 0, 1.0 / jnp.sqrt(jnp.maximum(deg_out, 1.0)), 0.0)
    nd_ref[...] = jnp.where(
        deg_in > 0, 1.0 / jnp.sqrt(jnp.maximum(deg_in, 1.0)), 0.0)


def _norms(degp2):
    return pl.pallas_call(
        _norm_body,
        out_shape=[
            jax.ShapeDtypeStruct((NP,), jnp.float32),
            jax.ShapeDtypeStruct((NP,), jnp.float32),
        ],
    )(degp2)


def _dense_body(x_ref, w_ref, ns_ref, g_ref):
    h = jnp.dot(x_ref[...], w_ref[...], preferred_element_type=jnp.float32)
    g_ref[...] = h * ns_ref[...][:, None]


def _dense(x_p, W, ns):
    return pl.pallas_call(
        _dense_body,
        grid=(NR,),
        in_specs=[
            pl.BlockSpec((128, D), lambda i: (i, 0)),
            pl.BlockSpec((D, D), lambda i: (0, 0)),
            pl.BlockSpec((128,), lambda i: (i,)),
        ],
        out_specs=pl.BlockSpec((128, D), lambda i: (i, 0)),
        out_shape=jax.ShapeDtypeStruct((NP, D), jnp.float32),
    )(x_p, W, ns)


@functools.partial(
    pl.kernel,
    mesh=_mesh,
    out_type=jax.ShapeDtypeStruct((NC, NP, D), jnp.float32),
    scratch_types=[
        pltpu.VMEM((CH, CHUNK), jnp.int32),
        pltpu.VMEM((CH, CHUNK), jnp.int32),
        pltpu.VMEM((CHUNK, D), jnp.float32),
        pltpu.VMEM_SHARED((NP, D), jnp.float32),
        pltpu.SemaphoreType.DMA,
    ],
)
def _agg_kernel(g_h, src_h, dst_h, out_h, sidx, didx, buf, agg_sh, sem):
    cid = lax.axis_index("c")
    sid = lax.axis_index("s")
    wid = sid * NC + cid

    def _zrow(r, _):
        for cc in range(D // 16):
            buf[r, pl.ds(cc * 16, 16)] = jnp.zeros((16,), jnp.float32)
        return 0

    lax.fori_loop(0, CHUNK, _zrow, 0)

    pltpu.sync_copy(src_h.at[wid], sidx)
    pltpu.sync_copy(dst_h.at[wid], didx)

    row0 = sid * RPT
    for k in range(RPT // CHUNK):
        pltpu.sync_copy(buf, agg_sh.at[pl.ds(row0 + k * CHUNK, CHUNK)])
    _tail = RPT % CHUNK
    if _tail:
        pltpu.sync_copy(
            buf.at[pl.ds(0, _tail)],
            agg_sh.at[pl.ds(row0 + (RPT // CHUNK) * CHUNK, _tail)],
        )

    plsc.subcore_barrier()

    def _edge_chunk(j, _):
        pltpu.async_copy(g_h.at[sidx.at[j]], buf, sem).wait()
        pltpu.sync_copy(buf, agg_sh.at[didx.at[j]], add=True)
        return 0

    lax.fori_loop(0, CH, _edge_chunk, 0)

    plsc.subcore_barrier()

    for k in range(RPT // CHUNK):
        pltpu.sync_copy(agg_sh.at[pl.ds(row0 + k * CHUNK, CHUNK)], buf)
        pltpu.sync_copy(buf, out_h.at[cid, pl.ds(row0 + k * CHUNK, CHUNK)])
    if _tail:
        _t0 = row0 + (RPT // CHUNK) * CHUNK
        pltpu.sync_copy(agg_sh.at[pl.ds(_t0, _tail)], buf.at[pl.ds(0, _tail)])
        pltpu.sync_copy(buf.at[pl.ds(0, _tail)], out_h.at[cid, pl.ds(_t0, _tail)])


def _final_body(agg_ref, nd_ref, b_ref, out_ref):
    out_ref[...] = (
        (agg_ref[0] + agg_ref[1]) * nd_ref[...][:, None] + b_ref[...][None, :]
    )


def _final(agg, nd, b):
    return pl.pallas_call(
        _final_body,
        grid=(NR,),
        in_specs=[
            pl.BlockSpec((NC, 128, D), lambda i: (0, i, 0)),
            pl.BlockSpec((128,), lambda i: (i,)),
            pl.BlockSpec((D,), lambda i: (0,)),
        ],
        out_specs=pl.BlockSpec((128, D), lambda i: (i, 0)),
        out_shape=jax.ShapeDtypeStruct((NP, D), jnp.float32),
    )(agg, nd, b)


def kernel(x, edge_index, W, b):
    src = edge_index[0].astype(jnp.int32)
    dst = edge_index[1].astype(jnp.int32)
    pad = jnp.full((EP - E,), N, jnp.int32)
    srcp = jnp.concatenate([src, pad]).reshape(NW, CH, CHUNK)
    dstf = jnp.concatenate([dst, pad])
    dstp = dstf.reshape(NW, CH, CHUNK)
    dsto = (dstf + NP).reshape(NW, CH, CHUNK)

    degp = _deg_kernel(srcp, dsto)
    ns, nd = _norms(degp.reshape(NC * 2, NP))

    x_p = jnp.concatenate([x, jnp.zeros((NP - N, D), x.dtype)], axis=0)
    g = _dense(x_p, W, ns)

    agg = _agg_kernel(g, srcp, dstp)
    return _final(agg, nd, b)[:N]


# speedup vs baseline: 8.6129x; 1.1132x over previous
import functools

import jax
import jax.numpy as jnp
from jax import lax
from jax.experimental import pallas as pl
from jax.experimental.pallas import tpu as pltpu, tpu_sc as plsc

N = 10000
E = 320000
D = 128
NP = 10112
NR = NP // 128
NC = 2
NS = 16
NW = NC * NS
EP = 327680
EPW = EP // NW
CHUNK = 128
CH = EPW // CHUNK
PH = 2
CPP = CH // PH
RPT = NP // NS

_mesh = plsc.VectorSubcoreMesh(core_axis_name="c", subcore_axis_name="s")


@functools.partial(
    pl.kernel,
    mesh=_mesh,
    out_type=jax.ShapeDtypeStruct((NC * 2 * NP,), jnp.float32),
    scratch_types=[
        pltpu.VMEM((CH, CHUNK), jnp.int32),
        pltpu.VMEM((CH, CHUNK), jnp.int32),
        pltpu.VMEM((CHUNK,), jnp.float32),
        pltpu.VMEM((NP,), jnp.float32),
        pltpu.VMEM_SHARED((2 * NP,), jnp.float32),
    ],
)
def _deg_kernel(src_h, dst_h, out_h, sidx, didx, ones_v, stage_v, deg_sh):
    cid = lax.axis_index("c")
    sid = lax.axis_index("s")
    wid = sid * NC + cid

    def _fill_ones(i, _):
        ones_v[pl.ds(i * 16, 16)] = jnp.ones((16,), jnp.float32)
        return 0

    lax.fori_loop(0, CHUNK // 16, _fill_ones, 0)

    def _fill_zero(i, _):
        stage_v[pl.ds(i * 16, 16)] = jnp.zeros((16,), jnp.float32)
        return 0

    lax.fori_loop(0, NP // 16, _fill_zero, 0)

    pltpu.sync_copy(src_h.at[wid], sidx)
    pltpu.sync_copy(dst_h.at[wid], didx)

    @pl.when(sid == 0)
    def _():
        pltpu.sync_copy(stage_v, deg_sh.at[pl.ds(0, NP)])

    @pl.when(sid == 1)
    def _():
        pltpu.sync_copy(stage_v, deg_sh.at[pl.ds(NP, NP)])

    plsc.subcore_barrier()

    def _accum(j, _):
        pltpu.sync_copy(ones_v, deg_sh.at[sidx.at[j]], add=True)
        pltpu.sync_copy(ones_v, deg_sh.at[didx.at[j]], add=True)
        return 0

    lax.fori_loop(0, CH, _accum, 0)

    plsc.subcore_barrier()

    @pl.when(sid == 0)
    def _():
        pltpu.sync_copy(deg_sh.at[pl.ds(0, NP)], stage_v)
        pltpu.sync_copy(stage_v, out_h.at[pl.ds(cid * 2 * NP, NP)])

    @pl.when(sid == 1)
    def _():
        pltpu.sync_copy(deg_sh.at[pl.ds(NP, NP)], stage_v)
        pltpu.sync_copy(stage_v, out_h.at[pl.ds(cid * 2 * NP + NP, NP)])


def _norm_body(dp_ref, ns_ref, nd_ref):
    deg_out = dp_ref[0, :] + dp_ref[2, :]
    deg_in = dp_ref[1, :] + dp_ref[3, :]
    ns_ref[...] = jnp.where(
        deg_out > 0, 1.0 / jnp.sqrt(jnp.maximum(deg_out, 1.0)), 0.0)
    nd_ref[...] = jnp.where(
        deg_in > 0, 1.0 / jnp.sqrt(jnp.maximum(deg_in, 1.0)), 0.0)


def _norms(degp2):
    return pl.pallas_call(
        _norm_body,
        out_shape=[
            jax.ShapeDtypeStruct((NP,), jnp.float32),
            jax.ShapeDtypeStruct((NP,), jnp.float32),
        ],
    )(degp2)


def _dense_body(x_ref, w_ref, ns_ref, g_ref):
    h = jnp.dot(x_ref[...], w_ref[...], preferred_element_type=jnp.float32)
    g_ref[...] = h * ns_ref[...][:, None]


def _dense(x_p, W, ns):
    return pl.pallas_call(
        _dense_body,
        grid=(NR,),
        in_specs=[
            pl.BlockSpec((128, D), lambda i: (i, 0)),
            pl.BlockSpec((D, D), lambda i: (0, 0)),
            pl.BlockSpec((128,), lambda i: (i,)),
        ],
        out_specs=pl.BlockSpec((128, D), lambda i: (i, 0)),
        out_shape=jax.ShapeDtypeStruct((NP, D), jnp.float32),
    )(x_p, W, ns)


@functools.partial(
    pl.kernel,
    mesh=_mesh,
    out_type=jax.ShapeDtypeStruct((NC, NP, D), jnp.float32),
    scratch_types=[
        pltpu.VMEM((CPP, CHUNK), jnp.int32),
        pltpu.VMEM((CPP, CHUNK), jnp.int32),
        pltpu.VMEM((CHUNK, D), jnp.float32),
        pltpu.VMEM((CHUNK, D), jnp.float32),
        pltpu.VMEM_SHARED((NP, D), jnp.float32),
        pltpu.SemaphoreType.DMA,
        pltpu.SemaphoreType.DMA,
    ],
)
def _agg_kernel(g_h, src_h, dst_h, out_h, sidx, didx, buf, bufb, agg_sh, sem, semb):
    cid = lax.axis_index("c")
    sid = lax.axis_index("s")
    wid = sid * NC + cid

    def _zrow(r, _):
        for cc in range(D // 16):
            buf[r, pl.ds(cc * 16, 16)] = jnp.zeros((16,), jnp.float32)
        return 0

    lax.fori_loop(0, CHUNK, _zrow, 0)

    row0 = sid * RPT
    for k in range(RPT // CHUNK):
        pltpu.sync_copy(buf, agg_sh.at[pl.ds(row0 + k * CHUNK, CHUNK)])
    _tail = RPT % CHUNK
    if _tail:
        pltpu.sync_copy(
            buf.at[pl.ds(0, _tail)],
            agg_sh.at[pl.ds(row0 + (RPT // CHUNK) * CHUNK, _tail)],
        )

    plsc.subcore_barrier()

    for p in range(PH):
        pltpu.sync_copy(src_h.at[wid, pl.ds(p * CPP, CPP)], sidx)
        pltpu.sync_copy(dst_h.at[wid, pl.ds(p * CPP, CPP)], didx)
        pltpu.async_copy(g_h.at[sidx.at[0]], buf, sem)
        pltpu.async_copy(g_h.at[sidx.at[1]], bufb, semb)

        def _edge_pair(jj, _):
            j = jj * 2
            pltpu.make_async_copy(g_h.at[sidx.at[j]], buf, sem).wait()
            pltpu.sync_copy(buf, agg_sh.at[didx.at[j]], add=True)
            pltpu.async_copy(g_h.at[sidx.at[j + 2]], buf, sem)
            pltpu.make_async_copy(g_h.at[sidx.at[j + 1]], bufb, semb).wait()
            pltpu.sync_copy(bufb, agg_sh.at[didx.at[j + 1]], add=True)
            pltpu.async_copy(g_h.at[sidx.at[j + 3]], bufb, semb)
            return 0

        lax.fori_loop(0, CPP // 2 - 1, _edge_pair, 0)

        pltpu.make_async_copy(g_h.at[sidx.at[CPP - 2]], buf, sem).wait()
        pltpu.sync_copy(buf, agg_sh.at[didx.at[CPP - 2]], add=True)
        pltpu.make_async_copy(g_h.at[sidx.at[CPP - 1]], bufb, semb).wait()
        pltpu.sync_copy(bufb, agg_sh.at[didx.at[CPP - 1]], add=True)

    plsc.subcore_barrier()

    for k in range(RPT // CHUNK):
        pltpu.sync_copy(agg_sh.at[pl.ds(row0 + k * CHUNK, CHUNK)], buf)
        pltpu.sync_copy(buf, out_h.at[cid, pl.ds(row0 + k * CHUNK, CHUNK)])
    if _tail:
        _t0 = row0 + (RPT // CHUNK) * CHUNK
        pltpu.sync_copy(agg_sh.at[pl.ds(_t0, _tail)], buf.at[pl.ds(0, _tail)])
        pltpu.sync_copy(buf.at[pl.ds(0, _tail)], out_h.at[cid, pl.ds(_t0, _tail)])


def _final_body(agg_ref, nd_ref, b_ref, out_ref):
    out_ref[...] = (
        (agg_ref[0] + agg_ref[1]) * nd_ref[...][:, None] + b_ref[...][None, :]
    )


def _final(agg, nd, b):
    return pl.pallas_call(
        _final_body,
        grid=(NR,),
        in_specs=[
            pl.BlockSpec((NC, 128, D), lambda i: (0, i, 0)),
            pl.BlockSpec((128,), lambda i: (i,)),
            pl.BlockSpec((D,), lambda i: (0,)),
        ],
        out_specs=pl.BlockSpec((128, D), lambda i: (i, 0)),
        out_shape=jax.ShapeDtypeStruct((NP, D), jnp.float32),
    )(agg, nd, b)


def kernel(x, edge_index, W, b):
    src = edge_index[0].astype(jnp.int32)
    dst = edge_index[1].astype(jnp.int32)
    pad = jnp.full((EP - E,), N, jnp.int32)
    srcp = jnp.concatenate([src, pad]).reshape(NW, CH, CHUNK)
    dstf = jnp.concatenate([dst, pad])
    dstp = dstf.reshape(NW, CH, CHUNK)
    dsto = (dstf + NP).reshape(NW, CH, CHUNK)

    degp = _deg_kernel(srcp, dsto)
    ns, nd = _norms(degp.reshape(NC * 2, NP))

    x_p = jnp.concatenate([x, jnp.zeros((NP - N, D), x.dtype)], axis=0)
    g = _dense(x_p, W, ns)

    agg = _agg_kernel(g, srcp, dstp)
    return _final(agg, nd, b)[:N]


# speedup vs baseline: 13.0519x; 1.5154x over previous
import functools

import jax
import jax.numpy as jnp
from jax import lax
from jax.experimental import pallas as pl
from jax.experimental.pallas import tpu as pltpu, tpu_sc as plsc

N = 10000
E = 320000
D = 128
DH = D // 2
NP = 10112
NR = NP // 128
NC = 2
NS = 16
NW = NC * NS
EP = 327680
EPW = EP // NW
CHUNK = 128
CH = EPW // CHUNK
EPT = EP // NS
CHT = EPT // CHUNK
UNROLL = 8
NPH = CHT // UNROLL
RPT = NP // NS

_mesh = plsc.VectorSubcoreMesh(core_axis_name="c", subcore_axis_name="s")


@functools.partial(
    pl.kernel,
    mesh=_mesh,
    out_type=jax.ShapeDtypeStruct((NC * 2 * NP,), jnp.float32),
    scratch_types=[
        pltpu.VMEM((CH, CHUNK), jnp.int32),
        pltpu.VMEM((CH, CHUNK), jnp.int32),
        pltpu.VMEM((CHUNK,), jnp.float32),
        pltpu.VMEM((NP,), jnp.float32),
        pltpu.VMEM_SHARED((2 * NP,), jnp.float32),
    ],
)
def _deg_kernel(src_h, dst_h, out_h, sidx, didx, ones_v, stage_v, deg_sh):
    cid = lax.axis_index("c")
    sid = lax.axis_index("s")
    wid = sid * NC + cid

    def _fill_ones(i, _):
        ones_v[pl.ds(i * 16, 16)] = jnp.ones((16,), jnp.float32)
        return 0

    lax.fori_loop(0, CHUNK // 16, _fill_ones, 0)

    def _fill_zero(i, _):
        stage_v[pl.ds(i * 16, 16)] = jnp.zeros((16,), jnp.float32)
        return 0

    lax.fori_loop(0, NP // 16, _fill_zero, 0)

    pltpu.sync_copy(src_h.at[wid], sidx)
    pltpu.sync_copy(dst_h.at[wid], didx)

    @pl.when(sid == 0)
    def _():
        pltpu.sync_copy(stage_v, deg_sh.at[pl.ds(0, NP)])

    @pl.when(sid == 1)
    def _():
        pltpu.sync_copy(stage_v, deg_sh.at[pl.ds(NP, NP)])

    plsc.subcore_barrier()

    def _accum(j, _):
        pltpu.sync_copy(ones_v, deg_sh.at[sidx.at[j]], add=True)
        pltpu.sync_copy(ones_v, deg_sh.at[didx.at[j]], add=True)
        return 0

    lax.fori_loop(0, CH, _accum, 0)

    plsc.subcore_barrier()

    @pl.when(sid == 0)
    def _():
        pltpu.sync_copy(deg_sh.at[pl.ds(0, NP)], stage_v)
        pltpu.sync_copy(stage_v, out_h.at[pl.ds(cid * 2 * NP, NP)])

    @pl.when(sid == 1)
    def _():
        pltpu.sync_copy(deg_sh.at[pl.ds(NP, NP)], stage_v)
        pltpu.sync_copy(stage_v, out_h.at[pl.ds(cid * 2 * NP + NP, NP)])


def _norm_body(dp_ref, ns_ref, nd_ref):
    deg_out = dp_ref[0, :] + dp_ref[2, :]
    deg_in = dp_ref[1, :] + dp_ref[3, :]
    ns_ref[...] = jnp.where(
        deg_out > 0, 1.0 / jnp.sqrt(jnp.maximum(deg_out, 1.0)), 0.0)
    nd_ref[...] = jnp.where(
        deg_in > 0, 1.0 / jnp.sqrt(jnp.maximum(deg_in, 1.0)), 0.0)


def _norms(degp2):
    return pl.pallas_call(
        _norm_body,
        out_shape=[
            jax.ShapeDtypeStruct((NP,), jnp.float32),
            jax.ShapeDtypeStruct((NP,), jnp.float32),
        ],
    )(degp2)


def _dense_body(x_ref, w_ref, ns_ref, g_ref):
    h = jnp.dot(x_ref[...], w_ref[...], preferred_element_type=jnp.float32)
    hs = h * ns_ref[...][:, None]
    g_ref[0] = hs[:, :DH]
    g_ref[1] = hs[:, DH:]


def _dense(x_p, W, ns):
    return pl.pallas_call(
        _dense_body,
        grid=(NR,),
        in_specs=[
            pl.BlockSpec((128, D), lambda i: (i, 0)),
            pl.BlockSpec((D, D), lambda i: (0, 0)),
            pl.BlockSpec((128,), lambda i: (i,)),
        ],
        out_specs=pl.BlockSpec((NC, 128, DH), lambda i: (0, i, 0)),
        out_shape=jax.ShapeDtypeStruct((NC, NP, DH), jnp.float32),
    )(x_p, W, ns)


@functools.partial(
    pl.kernel,
    mesh=_mesh,
    out_type=jax.ShapeDtypeStruct((NC, NP, DH), jnp.float32),
    scratch_types=[
        pltpu.VMEM((UNROLL, CHUNK), jnp.int32),
        pltpu.VMEM((UNROLL, CHUNK), jnp.int32),
        pltpu.VMEM((CHUNK, DH), jnp.float32),
        pltpu.VMEM((CHUNK, DH), jnp.float32),
        pltpu.VMEM_SHARED((NP, DH), jnp.float32),
        pltpu.VMEM_SHARED((NP, DH), jnp.float32),
        pltpu.SemaphoreType.DMA,
        pltpu.SemaphoreType.DMA,
    ],
    compiler_params=pltpu.CompilerParams(use_tc_tiling_on_sc=False),
)
def _agg_kernel(g_h, src_h, dst_h, out_h, sidx, didx, buf, bufb, g_sh, agg_sh,
                sem, semb):
    cid = lax.axis_index("c")
    sid = lax.axis_index("s")
    row0 = sid * RPT
    _tail = RPT % CHUNK

    for k in range(RPT // CHUNK):
        pltpu.sync_copy(g_h.at[cid, pl.ds(row0 + k * CHUNK, CHUNK)], buf)
        pltpu.sync_copy(buf, g_sh.at[pl.ds(row0 + k * CHUNK, CHUNK)])
    if _tail:
        _s0 = row0 + (RPT // CHUNK) * CHUNK
        pltpu.sync_copy(g_h.at[cid, pl.ds(_s0, _tail)], buf.at[pl.ds(0, _tail)])
        pltpu.sync_copy(buf.at[pl.ds(0, _tail)], g_sh.at[pl.ds(_s0, _tail)])

    def _zrow(r, _):
        for cc in range(DH // 16):
            buf[r, pl.ds(cc * 16, 16)] = jnp.zeros((16,), jnp.float32)
        return 0

    lax.fori_loop(0, CHUNK, _zrow, 0)

    for k in range(RPT // CHUNK):
        pltpu.sync_copy(buf, agg_sh.at[pl.ds(row0 + k * CHUNK, CHUNK)])
    if _tail:
        pltpu.sync_copy(
            buf.at[pl.ds(0, _tail)],
            agg_sh.at[pl.ds(row0 + (RPT // CHUNK) * CHUNK, _tail)],
        )

    plsc.subcore_barrier()

    bufs = (buf, bufb)
    sems = (sem, semb)

    def _phase(p, _):
        base = p * UNROLL
        pltpu.sync_copy(src_h.at[sid, pl.ds(base, UNROLL)], sidx)
        pltpu.sync_copy(dst_h.at[sid, pl.ds(base, UNROLL)], didx)
        descs = [pltpu.async_copy(g_sh.at[sidx.at[0]], bufs[0], sems[0])]
        for j in range(UNROLL):
            if j + 1 < UNROLL:
                descs.append(pltpu.async_copy(
                    g_sh.at[sidx.at[j + 1]], bufs[(j + 1) % 2], sems[(j + 1) % 2]))
            descs[j].wait()
            pltpu.sync_copy(bufs[j % 2], agg_sh.at[didx.at[j]], add=True)
        return 0

    lax.fori_loop(0, NPH, _phase, 0)

    plsc.subcore_barrier()

    for k in range(RPT // CHUNK):
        pltpu.sync_copy(agg_sh.at[pl.ds(row0 + k * CHUNK, CHUNK)], buf)
        pltpu.sync_copy(buf, out_h.at[cid, pl.ds(row0 + k * CHUNK, CHUNK)])
    if _tail:
        _t0 = row0 + (RPT // CHUNK) * CHUNK
        pltpu.sync_copy(agg_sh.at[pl.ds(_t0, _tail)], buf.at[pl.ds(0, _tail)])
        pltpu.sync_copy(buf.at[pl.ds(0, _tail)], out_h.at[cid, pl.ds(_t0, _tail)])


def _final_body(agg_ref, nd_ref, b_ref, out_ref):
    full = jnp.concatenate([agg_ref[0], agg_ref[1]], axis=1)
    out_ref[...] = full * nd_ref[...][:, None] + b_ref[...][None, :]


def _final(agg, nd, b):
    return pl.pallas_call(
        _final_body,
        grid=(NR,),
        in_specs=[
            pl.BlockSpec((NC, 128, DH), lambda i: (0, i, 0)),
            pl.BlockSpec((128,), lambda i: (i,)),
            pl.BlockSpec((D,), lambda i: (0,)),
        ],
        out_specs=pl.BlockSpec((128, D), lambda i: (i, 0)),
        out_shape=jax.ShapeDtypeStruct((NP, D), jnp.float32),
    )(agg, nd, b)


def kernel(x, edge_index, W, b):
    src = edge_index[0].astype(jnp.int32)
    dst = edge_index[1].astype(jnp.int32)
    pad = jnp.full((EP - E,), N, jnp.int32)
    srcf = jnp.concatenate([src, pad])
    dstf = jnp.concatenate([dst, pad])
    srcp = srcf.reshape(NW, CH, CHUNK)
    dsto = (dstf + NP).reshape(NW, CH, CHUNK)
    srct = srcf.reshape(NS, CHT, CHUNK)
    srcc = jnp.stack([srct, srct + NP])
    dstt = dstf.reshape(NS, CHT, CHUNK)

    degp = _deg_kernel(srcp, dsto)
    ns, nd = _norms(degp.reshape(NC * 2, NP))

    x_p = jnp.concatenate([x, jnp.zeros((NP - N, D), x.dtype)], axis=0)
    g2 = _dense(x_p, W, ns)

    agg = _agg_kernel(g2, srct, dstt)
    return _final(agg, nd, b)[:N]


# speedup vs baseline: 17.7421x; 1.3594x over previous
import functools

import jax
import jax.numpy as jnp
from jax import lax
from jax.experimental import pallas as pl
from jax.experimental.pallas import tpu as pltpu, tpu_sc as plsc

N = 10000
E = 320000
D = 128
DH = D // 2
NP = 10112
NR = NP // 128
NC = 2
NS = 16
NW = NC * NS
EP = 327680
EPW = EP // NW
CHUNK = 128
CH = EPW // CHUNK
EPT = EP // NS
CHT = EPT // CHUNK
UNROLL = 8
NPH = CHT // UNROLL
RPT = NP // NS

_mesh = plsc.VectorSubcoreMesh(core_axis_name="c", subcore_axis_name="s")


@functools.partial(
    pl.kernel,
    mesh=_mesh,
    out_type=jax.ShapeDtypeStruct((NC * 2 * NP,), jnp.float32),
    scratch_types=[
        pltpu.VMEM((CH, CHUNK), jnp.int32),
        pltpu.VMEM((CH, CHUNK), jnp.int32),
        pltpu.VMEM((CHUNK,), jnp.float32),
        pltpu.VMEM((NP,), jnp.float32),
        pltpu.VMEM_SHARED((2 * NP,), jnp.float32),
    ],
)
def _deg_kernel(src_h, dst_h, out_h, sidx, didx, ones_v, stage_v, deg_sh):
    cid = lax.axis_index("c")
    sid = lax.axis_index("s")
    wid = sid * NC + cid

    def _fill_ones(i, _):
        ones_v[pl.ds(i * 16, 16)] = jnp.ones((16,), jnp.float32)
        return 0

    lax.fori_loop(0, CHUNK // 16, _fill_ones, 0)

    def _fill_zero(i, _):
        stage_v[pl.ds(i * 16, 16)] = jnp.zeros((16,), jnp.float32)
        return 0

    lax.fori_loop(0, NP // 16, _fill_zero, 0)

    pltpu.sync_copy(src_h.at[wid], sidx)
    pltpu.sync_copy(dst_h.at[wid], didx)

    @pl.when(sid == 0)
    def _():
        pltpu.sync_copy(stage_v, deg_sh.at[pl.ds(0, NP)])

    @pl.when(sid == 1)
    def _():
        pltpu.sync_copy(stage_v, deg_sh.at[pl.ds(NP, NP)])

    plsc.subcore_barrier()

    def _accum(j, _):
        pltpu.sync_copy(ones_v, deg_sh.at[sidx.at[j]], add=True)
        pltpu.sync_copy(ones_v, deg_sh.at[didx.at[j]], add=True)
        return 0

    lax.fori_loop(0, CH, _accum, 0)

    plsc.subcore_barrier()

    @pl.when(sid == 0)
    def _():
        pltpu.sync_copy(deg_sh.at[pl.ds(0, NP)], stage_v)
        pltpu.sync_copy(stage_v, out_h.at[pl.ds(cid * 2 * NP, NP)])

    @pl.when(sid == 1)
    def _():
        pltpu.sync_copy(deg_sh.at[pl.ds(NP, NP)], stage_v)
        pltpu.sync_copy(stage_v, out_h.at[pl.ds(cid * 2 * NP + NP, NP)])


def _mm_body(x_ref, w_ref, h_ref):
    h_ref[...] = jnp.dot(x_ref[...], w_ref[...],
                         preferred_element_type=jnp.float32)


def _matmul(x, W):
    return pl.pallas_call(
        _mm_body,
        out_shape=jax.ShapeDtypeStruct((N, D), jnp.float32),
    )(x, W)


def _scale_body(h_ref, dp_ref, g_ref, nd_ref):
    deg_out = dp_ref[0, :] + dp_ref[2, :]
    deg_in = dp_ref[1, :] + dp_ref[3, :]
    ns = jnp.where(deg_out > 0, 1.0 / jnp.sqrt(jnp.maximum(deg_out, 1.0)), 0.0)
    nd_ref[...] = jnp.where(
        deg_in > 0, 1.0 / jnp.sqrt(jnp.maximum(deg_in, 1.0)), 0.0)
    hs = h_ref[...] * ns[:N, None]
    g_ref[0, pl.ds(0, N)] = hs[:, :DH]
    g_ref[1, pl.ds(0, N)] = hs[:, DH:]
    pad = jnp.zeros((NP - N, DH), jnp.float32)
    g_ref[0, pl.ds(N, NP - N)] = pad
    g_ref[1, pl.ds(N, NP - N)] = pad


def _scale(h, degp2):
    return pl.pallas_call(
        _scale_body,
        out_shape=[
            jax.ShapeDtypeStruct((NC, NP, DH), jnp.float32),
            jax.ShapeDtypeStruct((NP,), jnp.float32),
        ],
    )(h, degp2)


@functools.partial(
    pl.kernel,
    mesh=_mesh,
    out_type=jax.ShapeDtypeStruct((NC, NP, DH), jnp.float32),
    scratch_types=[
        pltpu.VMEM((UNROLL, CHUNK), jnp.int32),
        pltpu.VMEM((UNROLL, CHUNK), jnp.int32),
        pltpu.VMEM((UNROLL, CHUNK), jnp.int32),
        pltpu.VMEM((UNROLL, CHUNK), jnp.int32),
        pltpu.VMEM((CHUNK, DH), jnp.float32),
        pltpu.VMEM((CHUNK, DH), jnp.float32),
        pltpu.VMEM_SHARED((NP, DH), jnp.float32),
        pltpu.VMEM_SHARED((NP, DH), jnp.float32),
        pltpu.SemaphoreType.DMA,
        pltpu.SemaphoreType.DMA,
        pltpu.SemaphoreType.DMA,
        pltpu.SemaphoreType.DMA,
        pltpu.SemaphoreType.DMA,
    ],
    compiler_params=pltpu.CompilerParams(use_tc_tiling_on_sc=False),
)
def _agg_kernel(g_h, src_h, dst_h, out_h, sidxa, didxa, sidxb, didxb,
                buf, bufb, g_sh, agg_sh, gsem, gsemb, ssem, ssemb, stsem):
    cid = lax.axis_index("c")
    sid = lax.axis_index("s")
    row0 = sid * RPT
    _tail = RPT % CHUNK

    for k in range(RPT // CHUNK):
        pltpu.sync_copy(g_h.at[cid, pl.ds(row0 + k * CHUNK, CHUNK)], buf)
        pltpu.sync_copy(buf, g_sh.at[pl.ds(row0 + k * CHUNK, CHUNK)])
    if _tail:
        _s0 = row0 + (RPT // CHUNK) * CHUNK
        pltpu.sync_copy(g_h.at[cid, pl.ds(_s0, _tail)], buf.at[pl.ds(0, _tail)])
        pltpu.sync_copy(buf.at[pl.ds(0, _tail)], g_sh.at[pl.ds(_s0, _tail)])

    def _zrow(r, _):
        for cc in range(DH // 16):
            buf[r, pl.ds(cc * 16, 16)] = jnp.zeros((16,), jnp.float32)
        return 0

    lax.fori_loop(0, CHUNK, _zrow, 0)

    for k in range(RPT // CHUNK):
        pltpu.sync_copy(buf, agg_sh.at[pl.ds(row0 + k * CHUNK, CHUNK)])
    if _tail:
        pltpu.sync_copy(
            buf.at[pl.ds(0, _tail)],
            agg_sh.at[pl.ds(row0 + (RPT // CHUNK) * CHUNK, _tail)],
        )

    plsc.subcore_barrier()

    bufs = (buf, bufb)
    gsems = (gsem, gsemb)
    ssems = (ssem, ssemb)

    def _run_phase(sidx, didx):
        gd = [pltpu.async_copy(g_sh.at[sidx.at[0]], bufs[0], gsems[0])]
        sd = [None] * UNROLL
        for j in range(UNROLL):
            if j + 1 < UNROLL:
                if j >= 1:
                    sd[j - 1].wait()
                gd.append(pltpu.async_copy(
                    g_sh.at[sidx.at[j + 1]], bufs[(j + 1) % 2],
                    gsems[(j + 1) % 2]))
            gd[j].wait()
            sd[j] = pltpu.async_copy(
                bufs[j % 2], agg_sh.at[didx.at[j]], ssems[j % 2], add=True)
        sd[UNROLL - 2].wait()
        sd[UNROLL - 1].wait()

    pltpu.sync_copy(src_h.at[sid, pl.ds(0, UNROLL)], sidxa)
    pltpu.sync_copy(dst_h.at[sid, pl.ds(0, UNROLL)], didxa)

    def _phase_pair(pp, _):
        p = pp * 2
        b1 = (p + 1) * UNROLL
        s1 = pltpu.async_copy(src_h.at[sid, pl.ds(b1, UNROLL)], sidxb, stsem)
        s2 = pltpu.async_copy(dst_h.at[sid, pl.ds(b1, UNROLL)], didxb, stsem)
        _run_phase(sidxa, didxa)
        s1.wait()
        s2.wait()
        b2 = jnp.minimum((p + 2) * UNROLL, CHT - UNROLL)
        s3 = pltpu.async_copy(src_h.at[sid, pl.ds(b2, UNROLL)], sidxa, stsem)
        s4 = pltpu.async_copy(dst_h.at[sid, pl.ds(b2, UNROLL)], didxa, stsem)
        _run_phase(sidxb, didxb)
        s3.wait()
        s4.wait()
        return 0

    lax.fori_loop(0, NPH // 2, _phase_pair, 0)

    plsc.subcore_barrier()

    for k in range(RPT // CHUNK):
        pltpu.sync_copy(agg_sh.at[pl.ds(row0 + k * CHUNK, CHUNK)], buf)
        pltpu.sync_copy(buf, out_h.at[cid, pl.ds(row0 + k * CHUNK, CHUNK)])
    if _tail:
        _t0 = row0 + (RPT // CHUNK) * CHUNK
        pltpu.sync_copy(agg_sh.at[pl.ds(_t0, _tail)], buf.at[pl.ds(0, _tail)])
        pltpu.sync_copy(buf.at[pl.ds(0, _tail)], out_h.at[cid, pl.ds(_t0, _tail)])


def _final_body(agg_ref, nd_ref, b_ref, out_ref):
    full = jnp.concatenate(
        [agg_ref[0, pl.ds(0, N)], agg_ref[1, pl.ds(0, N)]], axis=1)
    nd = nd_ref[pl.ds(0, N)]
    out_ref[...] = full * nd[:, None] + b_ref[...][None, :]


def _final(agg, nd, b):
    return pl.pallas_call(
        _final_body,
        out_shape=jax.ShapeDtypeStruct((N, D), jnp.float32),
    )(agg, nd, b)


def kernel(x, edge_index, W, b):
    src = edge_index[0].astype(jnp.int32)
    dst = edge_index[1].astype(jnp.int32)
    pad = jnp.full((EP - E,), N, jnp.int32)
    srcf = jnp.concatenate([src, pad])
    dstf = jnp.concatenate([dst, pad])
    srcp = srcf.reshape(NW, CH, CHUNK)
    dsto = (dstf + NP).reshape(NW, CH, CHUNK)
    srct = srcf.reshape(NS, CHT, CHUNK)
    srcc = jnp.stack([srct, srct + NP])
    dstt = dstf.reshape(NS, CHT, CHUNK)

    degp = _deg_kernel(srcp, dsto)
    h = _matmul(x, W)
    g2, nd = _scale(h, degp.reshape(NC * 2, NP))

    agg = _agg_kernel(g2, srct, dstt)
    return _final(agg, nd, b)


# speedup vs baseline: 17.7685x; 1.0015x over previous
import functools

import jax
import jax.numpy as jnp
from jax import lax
from jax.experimental import pallas as pl
from jax.experimental.pallas import tpu as pltpu, tpu_sc as plsc

N = 10000
E = 320000
D = 128
DH = D // 2
NP = 10112
NR = NP // 128
NC = 2
NS = 16
NW = NC * NS
EP = 327680
EPW = EP // NW
CHUNK = 128
CH = EPW // CHUNK
EPT = EP // NS
CHT = EPT // CHUNK
UNROLL = 8
NPH = CHT // UNROLL
RPT = NP // NS

_mesh = plsc.VectorSubcoreMesh(core_axis_name="c", subcore_axis_name="s")


DPH = 8


@functools.partial(
    pl.kernel,
    mesh=_mesh,
    out_type=jax.ShapeDtypeStruct((NC * 2 * NP,), jnp.float32),
    scratch_types=[
        pltpu.VMEM((CH, CHUNK), jnp.int32),
        pltpu.VMEM((CH, CHUNK), jnp.int32),
        pltpu.VMEM((CHUNK,), jnp.float32),
        pltpu.VMEM((NP,), jnp.float32),
        pltpu.VMEM_SHARED((NP,), jnp.float32),
        pltpu.VMEM_SHARED((NP,), jnp.float32),
        pltpu.SemaphoreType.DMA,
    ],
    compiler_params=pltpu.CompilerParams(use_tc_tiling_on_sc=False),
)
def _deg_kernel(src_h, dst_h, out_h, sidx, didx, ones_v, stage_v,
                dout_sh, din_sh, dsem):
    cid = lax.axis_index("c")
    sid = lax.axis_index("s")

    def _fill_ones(i, _):
        ones_v[pl.ds(i * 16, 16)] = jnp.ones((16,), jnp.float32)
        return 0

    lax.fori_loop(0, CHUNK // 16, _fill_ones, 0)

    def _fill_zero(i, _):
        stage_v[pl.ds(i * 16, 16)] = jnp.zeros((16,), jnp.float32)
        return 0

    lax.fori_loop(0, NP // 16, _fill_zero, 0)

    pltpu.sync_copy(src_h.at[sid, pl.ds(cid * CH, CH)], sidx)
    pltpu.sync_copy(dst_h.at[sid, pl.ds(cid * CH, CH)], didx)

    @pl.when(sid == 0)
    def _():
        pltpu.sync_copy(stage_v, dout_sh)

    @pl.when(sid == 1)
    def _():
        pltpu.sync_copy(stage_v, din_sh)

    plsc.subcore_barrier()

    def _burst(p, _):
        descs = []
        for q in range(DPH):
            j = p * DPH + q
            descs.append(pltpu.async_copy(
                ones_v, dout_sh.at[sidx.at[j]], dsem, add=True))
            descs.append(pltpu.async_copy(
                ones_v, din_sh.at[didx.at[j]], dsem, add=True))
        for d in descs:
            d.wait()
        return 0

    lax.fori_loop(0, CH // DPH, _burst, 0)

    plsc.subcore_barrier()

    @pl.when(sid == 0)
    def _():
        pltpu.sync_copy(dout_sh, stage_v)
        pltpu.sync_copy(stage_v, out_h.at[pl.ds(cid * 2 * NP, NP)])

    @pl.when(sid == 1)
    def _():
        pltpu.sync_copy(din_sh, stage_v)
        pltpu.sync_copy(stage_v, out_h.at[pl.ds(cid * 2 * NP + NP, NP)])


def _mm_body(x_ref, w_ref, h_ref):
    h_ref[...] = jnp.dot(x_ref[...], w_ref[...],
                         preferred_element_type=jnp.float32)


def _matmul(x, W):
    return pl.pallas_call(
        _mm_body,
        out_shape=jax.ShapeDtypeStruct((N, D), jnp.float32),
    )(x, W)


def _scale_body(h_ref, dp_ref, g_ref, nd_ref):
    deg_out = dp_ref[0, :] + dp_ref[2, :]
    deg_in = dp_ref[1, :] + dp_ref[3, :]
    ns = jnp.where(deg_out > 0, 1.0 / jnp.sqrt(jnp.maximum(deg_out, 1.0)), 0.0)
    nd_ref[...] = jnp.where(
        deg_in > 0, 1.0 / jnp.sqrt(jnp.maximum(deg_in, 1.0)), 0.0)
    hs = h_ref[...] * ns[:N, None]
    g_ref[0, pl.ds(0, N)] = hs[:, :DH]
    g_ref[1, pl.ds(0, N)] = hs[:, DH:]
    pad = jnp.zeros((NP - N, DH), jnp.float32)
    g_ref[0, pl.ds(N, NP - N)] = pad
    g_ref[1, pl.ds(N, NP - N)] = pad


def _scale(h, degp2):
    return pl.pallas_call(
        _scale_body,
        out_shape=[
            jax.ShapeDtypeStruct((NC, NP, DH), jnp.float32),
            jax.ShapeDtypeStruct((NP,), jnp.float32),
        ],
    )(h, degp2)


@functools.partial(
    pl.kernel,
    mesh=_mesh,
    out_type=jax.ShapeDtypeStruct((NC, NP, DH), jnp.float32),
    scratch_types=[
        pltpu.VMEM((UNROLL, CHUNK), jnp.int32),
        pltpu.VMEM((UNROLL, CHUNK), jnp.int32),
        pltpu.VMEM((UNROLL, CHUNK), jnp.int32),
        pltpu.VMEM((UNROLL, CHUNK), jnp.int32),
        pltpu.VMEM((CHUNK, DH), jnp.float32),
        pltpu.VMEM((CHUNK, DH), jnp.float32),
        pltpu.VMEM_SHARED((NP, DH), jnp.float32),
        pltpu.VMEM_SHARED((NP, DH), jnp.float32),
        pltpu.SemaphoreType.DMA,
        pltpu.SemaphoreType.DMA,
        pltpu.SemaphoreType.DMA,
        pltpu.SemaphoreType.DMA,
        pltpu.SemaphoreType.DMA,
    ],
    compiler_params=pltpu.CompilerParams(use_tc_tiling_on_sc=False),
)
def _agg_kernel(g_h, src_h, dst_h, out_h, sidxa, didxa, sidxb, didxb,
                buf, bufb, g_sh, agg_sh, gsem, gsemb, ssem, ssemb, stsem):
    cid = lax.axis_index("c")
    sid = lax.axis_index("s")
    row0 = sid * RPT
    _tail = RPT % CHUNK

    for k in range(RPT // CHUNK):
        pltpu.sync_copy(g_h.at[cid, pl.ds(row0 + k * CHUNK, CHUNK)], buf)
        pltpu.sync_copy(buf, g_sh.at[pl.ds(row0 + k * CHUNK, CHUNK)])
    if _tail:
        _s0 = row0 + (RPT // CHUNK) * CHUNK
        pltpu.sync_copy(g_h.at[cid, pl.ds(_s0, _tail)], buf.at[pl.ds(0, _tail)])
        pltpu.sync_copy(buf.at[pl.ds(0, _tail)], g_sh.at[pl.ds(_s0, _tail)])

    def _zrow(r, _):
        for cc in range(DH // 16):
            buf[r, pl.ds(cc * 16, 16)] = jnp.zeros((16,), jnp.float32)
        return 0

    lax.fori_loop(0, CHUNK, _zrow, 0)

    for k in range(RPT // CHUNK):
        pltpu.sync_copy(buf, agg_sh.at[pl.ds(row0 + k * CHUNK, CHUNK)])
    if _tail:
        pltpu.sync_copy(
            buf.at[pl.ds(0, _tail)],
            agg_sh.at[pl.ds(row0 + (RPT // CHUNK) * CHUNK, _tail)],
        )

    plsc.subcore_barrier()

    bufs = (buf, bufb)
    gsems = (gsem, gsemb)
    ssems = (ssem, ssemb)

    def _run_phase(sidx, didx):
        gd = [pltpu.async_copy(g_sh.at[sidx.at[0]], bufs[0], gsems[0])]
        sd = [None] * UNROLL
        for j in range(UNROLL):
            if j + 1 < UNROLL:
                if j >= 1:
                    sd[j - 1].wait()
                gd.append(pltpu.async_copy(
                    g_sh.at[sidx.at[j + 1]], bufs[(j + 1) % 2],
                    gsems[(j + 1) % 2]))
            gd[j].wait()
            sd[j] = pltpu.async_copy(
                bufs[j % 2], agg_sh.at[didx.at[j]], ssems[j % 2], add=True)
        sd[UNROLL - 2].wait()
        sd[UNROLL - 1].wait()

    pltpu.sync_copy(src_h.at[sid, pl.ds(0, UNROLL)], sidxa)
    pltpu.sync_copy(dst_h.at[sid, pl.ds(0, UNROLL)], didxa)

    def _phase_pair(pp, _):
        p = pp * 2
        b1 = (p + 1) * UNROLL
        s1 = pltpu.async_copy(src_h.at[sid, pl.ds(b1, UNROLL)], sidxb, stsem)
        s2 = pltpu.async_copy(dst_h.at[sid, pl.ds(b1, UNROLL)], didxb, stsem)
        _run_phase(sidxa, didxa)
        s1.wait()
        s2.wait()
        b2 = jnp.minimum((p + 2) * UNROLL, CHT - UNROLL)
        s3 = pltpu.async_copy(src_h.at[sid, pl.ds(b2, UNROLL)], sidxa, stsem)
        s4 = pltpu.async_copy(dst_h.at[sid, pl.ds(b2, UNROLL)], didxa, stsem)
        _run_phase(sidxb, didxb)
        s3.wait()
        s4.wait()
        return 0

    lax.fori_loop(0, NPH // 2, _phase_pair, 0)

    plsc.subcore_barrier()

    for k in range(RPT // CHUNK):
        pltpu.sync_copy(agg_sh.at[pl.ds(row0 + k * CHUNK, CHUNK)], buf)
        pltpu.sync_copy(buf, out_h.at[cid, pl.ds(row0 + k * CHUNK, CHUNK)])
    if _tail:
        _t0 = row0 + (RPT // CHUNK) * CHUNK
        pltpu.sync_copy(agg_sh.at[pl.ds(_t0, _tail)], buf.at[pl.ds(0, _tail)])
        pltpu.sync_copy(buf.at[pl.ds(0, _tail)], out_h.at[cid, pl.ds(_t0, _tail)])


def _final_body(agg_ref, nd_ref, b_ref, out_ref):
    full = jnp.concatenate(
        [agg_ref[0, pl.ds(0, N)], agg_ref[1, pl.ds(0, N)]], axis=1)
    nd = nd_ref[pl.ds(0, N)]
    out_ref[...] = full * nd[:, None] + b_ref[...][None, :]


def _final(agg, nd, b):
    return pl.pallas_call(
        _final_body,
        out_shape=jax.ShapeDtypeStruct((N, D), jnp.float32),
    )(agg, nd, b)


def kernel(x, edge_index, W, b):
    src = edge_index[0].astype(jnp.int32)
    dst = edge_index[1].astype(jnp.int32)
    pad = jnp.full((EP - E,), N, jnp.int32)
    srct = jnp.concatenate([src, pad]).reshape(NS, CHT, CHUNK)
    dstt = jnp.concatenate([dst, pad]).reshape(NS, CHT, CHUNK)

    degp = _deg_kernel(srct, dstt)
    h = _matmul(x, W)
    g2, nd = _scale(h, degp.reshape(NC * 2, NP))

    agg = _agg_kernel(g2, srct, dstt)
    return _final(agg, nd, b)
